# Initial kernel scaffold; baseline (speedup 1.0000x reference)
import jax
import jax.numpy as jnp
from jax.experimental import pallas as pl


def kernel(x, edge_index, edge_attr, ln_gamma, ln_beta, nn_W, nn_b, root, bias):
    raise NotImplementedError("write your pallas kernel here")


# speedup vs baseline: 4.1121x; 4.1121x over previous
import functools

import jax
import jax.numpy as jnp
from jax import lax
from jax.experimental import pallas as pl
from jax.experimental.pallas import tpu as pltpu
from jax.experimental.pallas import tpu_sc as plsc

N, E, CIN, COUT, T = 10000, 160000, 128, 16, 16
ZC = (T + 1) * COUT
NP = 10240

NC, NS = 2, 16
NW = NC * NS
E_PER_W = E // NW
C = 40
NCHUNK = E_PER_W // C

ROWS = 1000


def _tc_body(x_ref, g_ref, b_ref, m2_ref, root_ref, bias_ref, z_ref, rt_ref):
    xb = x_ref[...]
    mu = jnp.mean(xb, axis=-1, keepdims=True)
    var = jnp.mean((xb - mu) ** 2, axis=-1, keepdims=True)
    h = (xb - mu) * lax.rsqrt(var + 1e-5) * g_ref[...] + b_ref[...]
    h = jnp.maximum(h, 0.0)
    z_ref[...] = jnp.dot(h, m2_ref[...], preferred_element_type=jnp.float32)
    rt_ref[...] = (
        jnp.dot(h, root_ref[...], preferred_element_type=jnp.float32)
        + bias_ref[...]
    )


def _tc_stage(x, ln_gamma, ln_beta, m2e, root, bias):
    grid = (N // ROWS,)
    return pl.pallas_call(
        _tc_body,
        grid=grid,
        in_specs=[
            pl.BlockSpec((ROWS, CIN), lambda i: (i, 0)),
            pl.BlockSpec((1, CIN), lambda i: (0, 0)),
            pl.BlockSpec((1, CIN), lambda i: (0, 0)),
            pl.BlockSpec((CIN, ZC), lambda i: (0, 0)),
            pl.BlockSpec((CIN, COUT), lambda i: (0, 0)),
            pl.BlockSpec((1, COUT), lambda i: (0, 0)),
        ],
        out_specs=[
            pl.BlockSpec((ROWS, ZC), lambda i: (i, 0)),
            pl.BlockSpec((ROWS, COUT), lambda i: (i, 0)),
        ],
        out_shape=[
            jax.ShapeDtypeStruct((N, ZC), jnp.float32),
            jax.ShapeDtypeStruct((N, COUT), jnp.float32),
        ],
    )(x, ln_gamma.reshape(1, CIN), ln_beta.reshape(1, CIN), m2e, root,
      bias.reshape(1, COUT))


def _sc_body(z_hbm, src_hbm, dst_hbm, ea_hbm, init_hbm, out_hbm,
             aggr_sh, srcix, dstix, ea_all, zrows, msg_buf, gsem):
    cid = lax.axis_index("c")
    sid = lax.axis_index("s")
    wid = cid * NS + sid

    @pl.when(sid == 0)
    def _():
        pltpu.sync_copy(init_hbm.at[cid], aggr_sh)

    plsc.subcore_barrier()

    pltpu.sync_copy(src_hbm.at[wid], srcix)
    pltpu.sync_copy(dst_hbm.at[wid], dstix)
    pltpu.sync_copy(ea_hbm.at[wid], ea_all)

    def chunk(k, carry):
        pltpu.async_copy(z_hbm.at[srcix.at[k]], zrows, gsem).wait()

        def edge(c, carry2):
            ea_vec = ea_all[k, c, :]
            acc = zrows[c, pl.ds(T * COUT, COUT)]
            for t in range(T):
                acc = acc + ea_vec[t] * zrows[c, pl.ds(t * COUT, COUT)]
            msg_buf[c, :] = acc
            return carry2

        lax.fori_loop(0, C, edge, 0, unroll=False)
        pltpu.sync_copy(msg_buf, aggr_sh.at[dstix.at[k]], add=True)
        return carry

    lax.fori_loop(0, NCHUNK, chunk, 0, unroll=False)

    plsc.subcore_barrier()

    rpt = NP // NS
    pltpu.sync_copy(aggr_sh.at[pl.ds(sid * rpt, rpt)],
                    out_hbm.at[cid, pl.ds(sid * rpt, rpt)])


_sc_stage = pl.kernel(
    _sc_body,
    out_type=jax.ShapeDtypeStruct((NC, NP, COUT), jnp.float32),
    mesh=plsc.VectorSubcoreMesh(core_axis_name="c", subcore_axis_name="s"),
    compiler_params=pltpu.CompilerParams(use_tc_tiling_on_sc=False),
    scratch_types=[
        pltpu.VMEM_SHARED((NP, COUT), jnp.float32),
        pltpu.VMEM((NCHUNK, C), jnp.int32),
        pltpu.VMEM((NCHUNK, C), jnp.int32),
        pltpu.VMEM((NCHUNK, C, T), jnp.float32),
        pltpu.VMEM((C, ZC), jnp.float32),
        pltpu.VMEM((C, COUT), jnp.float32),
        pltpu.SemaphoreType.DMA,
    ],
)


def kernel(x, edge_index, edge_attr, ln_gamma, ln_beta, nn_W, nn_b, root, bias):
    m2 = nn_W.reshape(T, CIN, COUT).transpose(1, 0, 2).reshape(CIN, T * COUT)
    m2e = jnp.concatenate([m2, nn_b.reshape(CIN, COUT)], axis=1)

    z, rt = _tc_stage(x, ln_gamma, ln_beta, m2e, root, bias)

    src = edge_index[0].reshape(NW, NCHUNK, C)
    dst = edge_index[1].reshape(NW, NCHUNK, C)
    ea = edge_attr.reshape(NW, NCHUNK, C, T)
    rt_pad = jnp.pad(rt, ((0, NP - N), (0, 0)))
    init = jnp.stack([rt_pad, jnp.zeros_like(rt_pad)])

    partial_sums = _sc_stage(z, src, dst, ea, init)
    return (partial_sums[0] + partial_sums[1])[:N]


# speedup vs baseline: 5.7280x; 1.3930x over previous
import functools

import jax
import jax.numpy as jnp
from jax import lax
from jax.experimental import pallas as pl
from jax.experimental.pallas import tpu as pltpu
from jax.experimental.pallas import tpu_sc as plsc

N, E, CIN, COUT, T = 10000, 160000, 128, 16, 16
ZC = (T + 1) * COUT
NP = 10240

NC, NS = 2, 16
NW = NC * NS
E_PER_W = E // NW
C = 100
NCHUNK = E_PER_W // C

ROWS = 1000


def _tc_body(x_ref, g_ref, b_ref, m2_ref, root_ref, bias_ref, z_ref, rt_ref):
    xb = x_ref[...]
    mu = jnp.mean(xb, axis=-1, keepdims=True)
    var = jnp.mean((xb - mu) ** 2, axis=-1, keepdims=True)
    h = (xb - mu) * lax.rsqrt(var + 1e-5) * g_ref[...] + b_ref[...]
    h = jnp.maximum(h, 0.0)
    z_ref[...] = jnp.dot(h, m2_ref[...], preferred_element_type=jnp.float32)
    rt_ref[...] = (
        jnp.dot(h, root_ref[...], preferred_element_type=jnp.float32)
        + bias_ref[...]
    )


def _tc_stage(x, ln_gamma, ln_beta, m2e, root, bias):
    grid = (N // ROWS,)
    return pl.pallas_call(
        _tc_body,
        grid=grid,
        in_specs=[
            pl.BlockSpec((ROWS, CIN), lambda i: (i, 0)),
            pl.BlockSpec((1, CIN), lambda i: (0, 0)),
            pl.BlockSpec((1, CIN), lambda i: (0, 0)),
            pl.BlockSpec((CIN, ZC), lambda i: (0, 0)),
            pl.BlockSpec((CIN, COUT), lambda i: (0, 0)),
            pl.BlockSpec((1, COUT), lambda i: (0, 0)),
        ],
        out_specs=[
            pl.BlockSpec((ROWS, ZC), lambda i: (i, 0)),
            pl.BlockSpec((ROWS, COUT), lambda i: (i, 0)),
        ],
        out_shape=[
            jax.ShapeDtypeStruct((N, ZC), jnp.float32),
            jax.ShapeDtypeStruct((N, COUT), jnp.float32),
        ],
    )(x, ln_gamma.reshape(1, CIN), ln_beta.reshape(1, CIN), m2e, root,
      bias.reshape(1, COUT))


def _sc_body(z_hbm, src_hbm, dst_hbm, ea_hbm, init_hbm, out_hbm,
             aggr_sh, srcix, dstix, ea0, ea1, zr0, zr1, msg0, msg1,
             gsem0, gsem1, esem0, esem1):
    cid = lax.axis_index("c")
    sid = lax.axis_index("s")
    wid = cid * NS + sid

    @pl.when(sid == 0)
    def _():
        pltpu.sync_copy(init_hbm.at[cid], aggr_sh)

    plsc.subcore_barrier()

    pltpu.sync_copy(src_hbm.at[wid], srcix)
    pltpu.sync_copy(dst_hbm.at[wid], dstix)

    def fetch(k, zr, gsem, ea, esem):
        pltpu.async_copy(z_hbm.at[srcix.at[k]], zr, gsem)
        pltpu.async_copy(ea_hbm.at[wid, k], ea, esem)

    def drain(k, zr, gsem, ea, esem):
        pltpu.make_async_copy(z_hbm.at[srcix.at[k]], zr, gsem).wait()
        pltpu.make_async_copy(ea_hbm.at[wid, k], ea, esem).wait()

    def compute(k, zr, ea, msg):
        def edge(c, carry2):
            ea_vec = ea[c, :]
            acc = zr[c, pl.ds(T * COUT, COUT)]
            for t in range(T):
                acc = acc + ea_vec[t] * zr[c, pl.ds(t * COUT, COUT)]
            msg[c, :] = acc
            return carry2

        lax.fori_loop(0, C, edge, 0, unroll=False)
        pltpu.sync_copy(msg, aggr_sh.at[dstix.at[k]], add=True)

    fetch(0, zr0, gsem0, ea0, esem0)
    npair = NCHUNK // 2

    def pair(j, carry):
        a = 2 * j
        fetch(a + 1, zr1, gsem1, ea1, esem1)
        drain(a, zr0, gsem0, ea0, esem0)
        compute(a, zr0, ea0, msg0)

        @pl.when(j < npair - 1)
        def _():
            fetch(a + 2, zr0, gsem0, ea0, esem0)

        drain(a + 1, zr1, gsem1, ea1, esem1)
        compute(a + 1, zr1, ea1, msg1)
        return carry

    lax.fori_loop(0, npair, pair, 0, unroll=False)

    plsc.subcore_barrier()

    rpt = NP // NS
    pltpu.sync_copy(aggr_sh.at[pl.ds(sid * rpt, rpt)],
                    out_hbm.at[cid, pl.ds(sid * rpt, rpt)])


_sc_stage = pl.kernel(
    _sc_body,
    out_type=jax.ShapeDtypeStruct((NC, NP, COUT), jnp.float32),
    mesh=plsc.VectorSubcoreMesh(core_axis_name="c", subcore_axis_name="s"),
    compiler_params=pltpu.CompilerParams(use_tc_tiling_on_sc=False),
    scratch_types=[
        pltpu.VMEM_SHARED((NP, COUT), jnp.float32),
        pltpu.VMEM((NCHUNK, C), jnp.int32),
        pltpu.VMEM((NCHUNK, C), jnp.int32),
        pltpu.VMEM((C, T), jnp.float32),
        pltpu.VMEM((C, T), jnp.float32),
        pltpu.VMEM((C, ZC), jnp.float32),
        pltpu.VMEM((C, ZC), jnp.float32),
        pltpu.VMEM((C, COUT), jnp.float32),
        pltpu.VMEM((C, COUT), jnp.float32),
        pltpu.SemaphoreType.DMA,
        pltpu.SemaphoreType.DMA,
        pltpu.SemaphoreType.DMA,
        pltpu.SemaphoreType.DMA,
    ],
)


def kernel(x, edge_index, edge_attr, ln_gamma, ln_beta, nn_W, nn_b, root, bias):
    m2 = nn_W.reshape(T, CIN, COUT).transpose(1, 0, 2).reshape(CIN, T * COUT)
    m2e = jnp.concatenate([m2, nn_b.reshape(CIN, COUT)], axis=1)

    z, rt = _tc_stage(x, ln_gamma, ln_beta, m2e, root, bias)

    src = edge_index[0].reshape(NW, NCHUNK, C)
    dst = edge_index[1].reshape(NW, NCHUNK, C)
    ea = edge_attr.reshape(NW, NCHUNK, C, T)
    rt_pad = jnp.pad(rt, ((0, NP - N), (0, 0)))
    init = jnp.stack([rt_pad, jnp.zeros_like(rt_pad)])

    partial_sums = _sc_stage(z, src, dst, ea, init)
    return (partial_sums[0] + partial_sums[1])[:N]


# speedup vs baseline: 6.6576x; 1.1623x over previous
import functools

import jax
import jax.numpy as jnp
from jax import lax
from jax.experimental import pallas as pl
from jax.experimental.pallas import tpu as pltpu
from jax.experimental.pallas import tpu_sc as plsc

N, E, CIN, COUT, T = 10000, 160000, 128, 16, 16
ZC = (T + 1) * COUT
NP = 10240

NC, NS = 2, 16
NW = NC * NS
E_PER_W = E // NW
C = 100
NCHUNK = E_PER_W // C
ZROWS = 128

ROWS = 1000


def _tc_body(x_ref, g_ref, b_ref, m2_ref, root_ref, bias_ref, z_ref, rt_ref):
    xb = x_ref[...]
    mu = jnp.mean(xb, axis=-1, keepdims=True)
    var = jnp.mean((xb - mu) ** 2, axis=-1, keepdims=True)
    h = (xb - mu) * lax.rsqrt(var + 1e-5) * g_ref[...] + b_ref[...]
    h = jnp.maximum(h, 0.0)
    z_ref[...] = jnp.dot(h, m2_ref[...], preferred_element_type=jnp.float32)
    rt_ref[...] = (
        jnp.dot(h, root_ref[...], preferred_element_type=jnp.float32)
        + bias_ref[...]
    )


def _tc_stage(x, ln_gamma, ln_beta, m2e, root, bias):
    grid = (N // ROWS,)
    return pl.pallas_call(
        _tc_body,
        grid=grid,
        in_specs=[
            pl.BlockSpec((ROWS, CIN), lambda i: (i, 0)),
            pl.BlockSpec((1, CIN), lambda i: (0, 0)),
            pl.BlockSpec((1, CIN), lambda i: (0, 0)),
            pl.BlockSpec((CIN, ZC), lambda i: (0, 0)),
            pl.BlockSpec((CIN, COUT), lambda i: (0, 0)),
            pl.BlockSpec((1, COUT), lambda i: (0, 0)),
        ],
        out_specs=[
            pl.BlockSpec((ROWS, ZC), lambda i: (i, 0)),
            pl.BlockSpec((ROWS, COUT), lambda i: (i, 0)),
        ],
        out_shape=[
            jax.ShapeDtypeStruct((N, ZC), jnp.float32),
            jax.ShapeDtypeStruct((N, COUT), jnp.float32),
        ],
    )(x, ln_gamma.reshape(1, CIN), ln_beta.reshape(1, CIN), m2e, root,
      bias.reshape(1, COUT))


def _sc_body(z_hbm, eidx_hbm, ea_hbm, rt_hbm, out_hbm,
             aggr_sh, srcix, dstix, ea0, ea1, zr0, zr1, msg0, msg1, zbuf,
             gsem0, gsem1, esem0, esem1):
    cid = lax.axis_index("c")
    sid = lax.axis_index("s")
    wid = cid * NS + sid
    rpt = NP // NS

    def zrow(i, carry):
        zbuf[i, :] = jnp.zeros((COUT,), jnp.float32)
        return carry

    lax.fori_loop(0, ZROWS, zrow, 0, unroll=False)
    for m in range(rpt // ZROWS):
        pltpu.sync_copy(zbuf,
                        aggr_sh.at[pl.ds(sid * rpt + m * ZROWS, ZROWS)])

    @pl.when(cid == 0)
    def _():
        @pl.when(sid < NS - 1)
        def _():
            pltpu.sync_copy(rt_hbm.at[pl.ds(sid * rpt, rpt)],
                            aggr_sh.at[pl.ds(sid * rpt, rpt)])

        @pl.when(sid == NS - 1)
        def _():
            pltpu.sync_copy(rt_hbm.at[pl.ds((NS - 1) * rpt, N - (NS - 1) * rpt)],
                            aggr_sh.at[pl.ds((NS - 1) * rpt, N - (NS - 1) * rpt)])

    pltpu.sync_copy(eidx_hbm.at[0, wid], srcix)
    pltpu.sync_copy(eidx_hbm.at[1, wid], dstix)

    def fetch(k, zr, gsem, ea, esem):
        pltpu.async_copy(z_hbm.at[srcix.at[k]], zr, gsem)
        pltpu.async_copy(ea_hbm.at[pl.ds((wid * NCHUNK + k) * C, C)], ea, esem)

    def drain(k, zr, gsem, ea, esem):
        pltpu.make_async_copy(z_hbm.at[srcix.at[k]], zr, gsem).wait()
        pltpu.make_async_copy(ea_hbm.at[pl.ds((wid * NCHUNK + k) * C, C)], ea,
                              esem).wait()

    def compute(k, zr, ea, msg):
        def edge(c, carry2):
            ea_vec = ea[c, :]
            acc = zr[c, pl.ds(T * COUT, COUT)]
            for t in range(T):
                acc = acc + ea_vec[t] * zr[c, pl.ds(t * COUT, COUT)]
            msg[c, :] = acc
            return carry2

        lax.fori_loop(0, C, edge, 0, unroll=False)
        pltpu.sync_copy(msg, aggr_sh.at[dstix.at[k]], add=True)

    fetch(0, zr0, gsem0, ea0, esem0)
    plsc.subcore_barrier()
    npair = NCHUNK // 2

    def pair(j, carry):
        a = 2 * j
        fetch(a + 1, zr1, gsem1, ea1, esem1)
        drain(a, zr0, gsem0, ea0, esem0)
        compute(a, zr0, ea0, msg0)

        @pl.when(j < npair - 1)
        def _():
            fetch(a + 2, zr0, gsem0, ea0, esem0)

        drain(a + 1, zr1, gsem1, ea1, esem1)
        compute(a + 1, zr1, ea1, msg1)
        return carry

    lax.fori_loop(0, npair, pair, 0, unroll=False)

    plsc.subcore_barrier()

    pltpu.sync_copy(aggr_sh.at[pl.ds(sid * rpt, rpt)],
                    out_hbm.at[cid, pl.ds(sid * rpt, rpt)])


_sc_stage = pl.kernel(
    _sc_body,
    out_type=jax.ShapeDtypeStruct((NC, NP, COUT), jnp.float32),
    mesh=plsc.VectorSubcoreMesh(core_axis_name="c", subcore_axis_name="s"),
    compiler_params=pltpu.CompilerParams(use_tc_tiling_on_sc=False),
    scratch_types=[
        pltpu.VMEM_SHARED((NP, COUT), jnp.float32),
        pltpu.VMEM((NCHUNK, C), jnp.int32),
        pltpu.VMEM((NCHUNK, C), jnp.int32),
        pltpu.VMEM((C, T), jnp.float32),
        pltpu.VMEM((C, T), jnp.float32),
        pltpu.VMEM((C, ZC), jnp.float32),
        pltpu.VMEM((C, ZC), jnp.float32),
        pltpu.VMEM((C, COUT), jnp.float32),
        pltpu.VMEM((C, COUT), jnp.float32),
        pltpu.VMEM((ZROWS, COUT), jnp.float32),
        pltpu.SemaphoreType.DMA,
        pltpu.SemaphoreType.DMA,
        pltpu.SemaphoreType.DMA,
        pltpu.SemaphoreType.DMA,
    ],
)


def kernel(x, edge_index, edge_attr, ln_gamma, ln_beta, nn_W, nn_b, root, bias):
    m2 = nn_W.reshape(T, CIN, COUT).transpose(1, 0, 2).reshape(CIN, T * COUT)
    m2e = jnp.concatenate([m2, nn_b.reshape(CIN, COUT)], axis=1)

    z, rt = _tc_stage(x, ln_gamma, ln_beta, m2e, root, bias)

    eidx = edge_index.reshape(2, NW, NCHUNK, C)
    partial_sums = _sc_stage(z, eidx, edge_attr, rt)
    return (partial_sums[0] + partial_sums[1])[:N]
